# baseline (speedup 1.0000x reference)
import functools

import jax
import jax.numpy as jnp
from jax import lax
from jax.experimental import pallas as pl
from jax.experimental.pallas import tpu as pltpu
from jax.experimental.pallas import tpu_sc as plsc

N_SC = 2
N_TILES = 16
NW = N_SC * N_TILES
CHUNK = 128
GRP = 16
NPAD = 10112
M0 = 320
M1 = 0
MTOT = M0 + M1


def _k1_body(x_ref, w_ref, degp_ref, o_ref):
    deg = degp_ref[0, :, 0] + degp_ref[1, :, 0] + 1.0
    dis = lax.rsqrt(deg)
    xw = jnp.dot(x_ref[...], w_ref[...], preferred_element_type=jnp.float32)
    o_ref[...] = dis[:, None] * xw


def _k2_body(p_ref, y1_ref, degp_ref, b1_ref, w2_ref, o_ref):
    deg = degp_ref[0, :, 0] + degp_ref[1, :, 0] + 1.0
    dis = lax.rsqrt(deg)
    s = p_ref[0] + p_ref[1] + y1_ref[...]
    h = jnp.maximum(dis[:, None] * s + b1_ref[...], 0.0)
    hw = jnp.dot(h, w2_ref[...], preferred_element_type=jnp.float32)
    o_ref[...] = jnp.concatenate(
        [dis[:, None] * hw, jnp.zeros_like(hw)], axis=1)


def _k3_body(q_ref, y2_ref, degp_ref, b2_ref, o_ref):
    deg = degp_ref[0, :, 0] + degp_ref[1, :, 0] + 1.0
    dis = lax.rsqrt(deg)
    nc = o_ref.shape[1]
    o = (dis[:, None] * (q_ref[0, :, :nc] + q_ref[1, :, :nc] + y2_ref[:, :nc])
         + b2_ref[...])
    m = jnp.max(o, axis=1, keepdims=True)
    lse = jnp.log(jnp.sum(jnp.exp(o - m), axis=1, keepdims=True)) + m
    o_ref[...] = o - lse


def _deg_body(dst_hbm, ones_hbm, zeros_hbm, out_hbm, dst_g, ones_v, deg_sh):
    c = lax.axis_index("c")
    s = lax.axis_index("s")
    w = c * N_TILES + s
    mpw = dst_hbm.shape[0] // NW
    off = w * mpw
    pltpu.sync_copy(ones_hbm, ones_v)
    pltpu.sync_copy(zeros_hbm, deg_sh.at[pl.ds(s * (NPAD // N_TILES), NPAD // N_TILES)])
    plsc.subcore_barrier()

    def group(g, carry):
        pltpu.sync_copy(dst_hbm.at[pl.ds(off + g * GRP, GRP)], dst_g)
        for j in range(GRP):
            pltpu.sync_copy(ones_v, deg_sh.at[dst_g.at[j]], add=True)
        return carry

    lax.fori_loop(0, mpw // GRP, group, 0, unroll=False)
    plsc.subcore_barrier()
    npt = NPAD // N_TILES
    pltpu.sync_copy(deg_sh.at[pl.ds(s * npt, npt)], out_hbm.at[c, pl.ds(s * npt, npt)])


def _agg_body(y_hbm, src_hbm, dst_hbm, zeros_hbm, out_hbm,
              src_g, dst_g, rows0, rows1, agg_sh,
              gsem0, gsem1, ssem0, ssem1):
    c = lax.axis_index("c")
    s = lax.axis_index("s")
    rows = (rows0, rows1)
    gsem = (gsem0, gsem1)
    ssem = (ssem0, ssem1)
    pltpu.sync_copy(zeros_hbm, agg_sh.at[pl.ds(s * (NPAD // N_TILES), NPAD // N_TILES)])
    plsc.subcore_barrier()

    def run(off, ngroups):
        def group(g, carry):
            base = off + g * GRP
            pltpu.sync_copy(src_hbm.at[pl.ds(base, GRP)], src_g)
            pltpu.sync_copy(dst_hbm.at[pl.ds(base, GRP)], dst_g)
            gd = [None, None]
            sd = [None, None]
            gd[0] = pltpu.async_copy(y_hbm.at[src_g.at[0]], rows[0], gsem[0])
            for j in range(GRP):
                b = j & 1
                nb = 1 - b
                if j + 1 < GRP:
                    if sd[nb] is not None:
                        sd[nb].wait()
                    gd[nb] = pltpu.async_copy(
                        y_hbm.at[src_g.at[j + 1]], rows[nb], gsem[nb])
                gd[b].wait()
                sd[b] = pltpu.async_copy(
                    rows[b], agg_sh.at[dst_g.at[j]], ssem[b], add=True)
            sd[(GRP - 2) & 1].wait()
            sd[(GRP - 1) & 1].wait()
            return carry
        lax.fori_loop(0, ngroups, group, 0, unroll=False)

    @pl.when(c == 0)
    def _():
        run(s * M0, M0 // GRP)

    @pl.when(c == 1)
    def _():
        run(N_TILES * M0 + s * M1, M1 // GRP)

    plsc.subcore_barrier()
    npt = NPAD // N_TILES
    pltpu.sync_copy(agg_sh.at[pl.ds(s * npt, npt)], out_hbm.at[c, pl.ds(s * npt, npt)])


@functools.cache
def _sc_mesh():
    return plsc.VectorSubcoreMesh(
        core_axis_name="c", subcore_axis_name="s",
        num_cores=N_SC, num_subcores=N_TILES)


def _deg_partials(dst_p):
    ones_rows = jnp.zeros((CHUNK, 128), jnp.float32).at[:, 0].set(1.0)
    zeros = jnp.zeros((NPAD // N_TILES, 128), jnp.float32)
    return pl.kernel(
        _deg_body,
        out_type=jax.ShapeDtypeStruct((N_SC, NPAD, 128), jnp.float32),
        mesh=_sc_mesh(),
        scratch_types=[
            pltpu.VMEM((GRP, CHUNK), jnp.int32),
            pltpu.VMEM((CHUNK, 128), jnp.float32),
            pltpu.VMEM_SHARED((NPAD, 128), jnp.float32),
        ],
    )(dst_p, ones_rows, zeros)


def _aggregate(y, src_p, dst_p):
    d = y.shape[1]
    zeros = jnp.zeros((NPAD // N_TILES, d), jnp.float32)
    return pl.kernel(
        _agg_body,
        out_type=jax.ShapeDtypeStruct((N_SC, NPAD, d), jnp.float32),
        mesh=_sc_mesh(),
        scratch_types=[
            pltpu.VMEM((GRP, CHUNK), jnp.int32),
            pltpu.VMEM((GRP, CHUNK), jnp.int32),
            pltpu.VMEM((CHUNK, d), jnp.float32),
            pltpu.VMEM((CHUNK, d), jnp.float32),
            pltpu.VMEM_SHARED((NPAD, d), jnp.float32),
            pltpu.SemaphoreType.DMA,
            pltpu.SemaphoreType.DMA,
            pltpu.SemaphoreType.DMA,
            pltpu.SemaphoreType.DMA,
        ],
    )(y, src_p, dst_p, zeros)


def kernel(x, edge_index, W1, b1, W2, b2):
    n, _ = x.shape
    hidden = W1.shape[1]
    classes = W2.shape[1]
    e = edge_index.shape[1]
    bm = 400
    grid = (n // bm,)

    src = edge_index[0].astype(jnp.int32)
    dst = edge_index[1].astype(jnp.int32)
    totc = N_TILES * MTOT
    e_pad = totc * CHUNK
    src_p = jnp.concatenate(
        [src, jnp.zeros((e_pad - e,), jnp.int32)]).reshape(totc, CHUNK)
    dst_p = jnp.concatenate(
        [dst, jnp.full((e_pad - e,), n, jnp.int32)]).reshape(totc, CHUNK)

    degp = _deg_partials(dst_p)

    degp_spec = pl.BlockSpec((N_SC, bm, 128), lambda i: (0, i, 0))
    y1 = pl.pallas_call(
        _k1_body,
        grid=grid,
        in_specs=[
            pl.BlockSpec((bm, x.shape[1]), lambda i: (i, 0)),
            pl.BlockSpec(W1.shape, lambda i: (0, 0)),
            degp_spec,
        ],
        out_specs=pl.BlockSpec((bm, hidden), lambda i: (i, 0)),
        out_shape=jax.ShapeDtypeStruct((n, hidden), jnp.float32),
    )(x, W1, degp)

    p = _aggregate(y1, src_p, dst_p)

    y2 = pl.pallas_call(
        _k2_body,
        grid=grid,
        in_specs=[
            pl.BlockSpec((N_SC, bm, hidden), lambda i: (0, i, 0)),
            pl.BlockSpec((bm, hidden), lambda i: (i, 0)),
            degp_spec,
            pl.BlockSpec((1, hidden), lambda i: (0, 0)),
            pl.BlockSpec(W2.shape, lambda i: (0, 0)),
        ],
        out_specs=pl.BlockSpec((bm, 2 * classes), lambda i: (i, 0)),
        out_shape=jax.ShapeDtypeStruct((n, 2 * classes), jnp.float32),
    )(p, y1, degp, b1.reshape(1, -1), W2)

    q = _aggregate(y2, src_p, dst_p)

    out = pl.pallas_call(
        _k3_body,
        grid=grid,
        in_specs=[
            pl.BlockSpec((N_SC, bm, 2 * classes), lambda i: (0, i, 0)),
            pl.BlockSpec((bm, 2 * classes), lambda i: (i, 0)),
            degp_spec,
            pl.BlockSpec((1, classes), lambda i: (0, 0)),
        ],
        out_specs=pl.BlockSpec((bm, classes), lambda i: (i, 0)),
        out_shape=jax.ShapeDtypeStruct((n, classes), jnp.float32),
    )(q, y2, degp, b2.reshape(1, -1))
    return out


# speedup vs baseline: 1.2414x; 1.2414x over previous
import functools

import jax
import jax.numpy as jnp
from jax import lax
from jax.experimental import pallas as pl
from jax.experimental.pallas import tpu as pltpu
from jax.experimental.pallas import tpu_sc as plsc

N_SC = 2
N_TILES = 16
NW = N_SC * N_TILES
CHUNK = 128
GRP = 16
NPAD = 10112
M0 = 304
M1 = 16
MTOT = M0 + M1


def _k1_body(x_ref, w_ref, degp_ref, o_ref):
    deg = degp_ref[0, :, 0] + degp_ref[1, :, 0] + 1.0
    dis = lax.rsqrt(deg)
    xw = jnp.dot(x_ref[...], w_ref[...], preferred_element_type=jnp.float32)
    o_ref[...] = dis[:, None] * xw


def _k2_body(p_ref, y1_ref, degp_ref, b1_ref, w2_ref, o_ref):
    deg = degp_ref[0, :, 0] + degp_ref[1, :, 0] + 1.0
    dis = lax.rsqrt(deg)
    s = p_ref[0] + p_ref[1] + y1_ref[...]
    h = jnp.maximum(dis[:, None] * s + b1_ref[...], 0.0)
    hw = jnp.dot(h, w2_ref[...], preferred_element_type=jnp.float32)
    o_ref[...] = jnp.concatenate(
        [dis[:, None] * hw, jnp.zeros_like(hw)], axis=1)


def _k3_body(q_ref, y2_ref, degp_ref, b2_ref, o_ref):
    deg = degp_ref[0, :, 0] + degp_ref[1, :, 0] + 1.0
    dis = lax.rsqrt(deg)
    nc = o_ref.shape[1]
    o = (dis[:, None] * (q_ref[0, :, :nc] + q_ref[1, :, :nc] + y2_ref[:, :nc])
         + b2_ref[...])
    m = jnp.max(o, axis=1, keepdims=True)
    lse = jnp.log(jnp.sum(jnp.exp(o - m), axis=1, keepdims=True)) + m
    o_ref[...] = o - lse


def _deg_body(dst_hbm, ones_hbm, zeros_hbm, out_hbm, dst_g, ones_v, deg_sh):
    c = lax.axis_index("c")
    s = lax.axis_index("s")
    w = c * N_TILES + s
    mpw = dst_hbm.shape[0] // NW
    off = w * mpw
    pltpu.sync_copy(ones_hbm, ones_v)
    pltpu.sync_copy(zeros_hbm, deg_sh.at[pl.ds(s * (NPAD // N_TILES), NPAD // N_TILES)])
    plsc.subcore_barrier()

    def group(g, carry):
        pltpu.sync_copy(dst_hbm.at[pl.ds(off + g * GRP, GRP)], dst_g)
        for j in range(GRP):
            pltpu.sync_copy(ones_v, deg_sh.at[dst_g.at[j]], add=True)
        return carry

    lax.fori_loop(0, mpw // GRP, group, 0, unroll=False)
    plsc.subcore_barrier()
    npt = NPAD // N_TILES
    pltpu.sync_copy(deg_sh.at[pl.ds(s * npt, npt)], out_hbm.at[c, pl.ds(s * npt, npt)])


def _agg_body(y_hbm, src_hbm, dst_hbm, zeros_hbm, out_hbm,
              src_g, dst_g, rows0, rows1, agg_sh,
              gsem0, gsem1, ssem0, ssem1):
    c = lax.axis_index("c")
    s = lax.axis_index("s")
    rows = (rows0, rows1)
    gsem = (gsem0, gsem1)
    ssem = (ssem0, ssem1)
    pltpu.sync_copy(zeros_hbm, agg_sh.at[pl.ds(s * (NPAD // N_TILES), NPAD // N_TILES)])
    plsc.subcore_barrier()

    def run(off, ngroups):
        def group(g, carry):
            base = off + g * GRP
            pltpu.sync_copy(src_hbm.at[pl.ds(base, GRP)], src_g)
            pltpu.sync_copy(dst_hbm.at[pl.ds(base, GRP)], dst_g)
            gd = [None, None]
            sd = [None, None]
            gd[0] = pltpu.async_copy(y_hbm.at[src_g.at[0]], rows[0], gsem[0])
            for j in range(GRP):
                b = j & 1
                nb = 1 - b
                if j + 1 < GRP:
                    if sd[nb] is not None:
                        sd[nb].wait()
                    gd[nb] = pltpu.async_copy(
                        y_hbm.at[src_g.at[j + 1]], rows[nb], gsem[nb])
                gd[b].wait()
                sd[b] = pltpu.async_copy(
                    rows[b], agg_sh.at[dst_g.at[j]], ssem[b], add=True)
            sd[(GRP - 2) & 1].wait()
            sd[(GRP - 1) & 1].wait()
            return carry
        lax.fori_loop(0, ngroups, group, 0, unroll=False)

    @pl.when(c == 0)
    def _():
        run(s * M0, M0 // GRP)

    @pl.when(c == 1)
    def _():
        run(N_TILES * M0 + s * M1, M1 // GRP)

    plsc.subcore_barrier()
    npt = NPAD // N_TILES
    pltpu.sync_copy(agg_sh.at[pl.ds(s * npt, npt)], out_hbm.at[c, pl.ds(s * npt, npt)])


@functools.cache
def _sc_mesh():
    return plsc.VectorSubcoreMesh(
        core_axis_name="c", subcore_axis_name="s",
        num_cores=N_SC, num_subcores=N_TILES)


def _deg_partials(dst_p):
    ones_rows = jnp.zeros((CHUNK, 128), jnp.float32).at[:, 0].set(1.0)
    zeros = jnp.zeros((NPAD // N_TILES, 128), jnp.float32)
    return pl.kernel(
        _deg_body,
        out_type=jax.ShapeDtypeStruct((N_SC, NPAD, 128), jnp.float32),
        mesh=_sc_mesh(),
        scratch_types=[
            pltpu.VMEM((GRP, CHUNK), jnp.int32),
            pltpu.VMEM((CHUNK, 128), jnp.float32),
            pltpu.VMEM_SHARED((NPAD, 128), jnp.float32),
        ],
    )(dst_p, ones_rows, zeros)


def _aggregate(y, src_p, dst_p):
    d = y.shape[1]
    zeros = jnp.zeros((NPAD // N_TILES, d), jnp.float32)
    return pl.kernel(
        _agg_body,
        out_type=jax.ShapeDtypeStruct((N_SC, NPAD, d), jnp.float32),
        mesh=_sc_mesh(),
        scratch_types=[
            pltpu.VMEM((GRP, CHUNK), jnp.int32),
            pltpu.VMEM((GRP, CHUNK), jnp.int32),
            pltpu.VMEM((CHUNK, d), jnp.float32),
            pltpu.VMEM((CHUNK, d), jnp.float32),
            pltpu.VMEM_SHARED((NPAD, d), jnp.float32),
            pltpu.SemaphoreType.DMA,
            pltpu.SemaphoreType.DMA,
            pltpu.SemaphoreType.DMA,
            pltpu.SemaphoreType.DMA,
        ],
    )(y, src_p, dst_p, zeros)


def kernel(x, edge_index, W1, b1, W2, b2):
    n, _ = x.shape
    hidden = W1.shape[1]
    classes = W2.shape[1]
    e = edge_index.shape[1]
    bm = 400
    grid = (n // bm,)

    src = edge_index[0].astype(jnp.int32)
    dst = edge_index[1].astype(jnp.int32)
    totc = N_TILES * MTOT
    e_pad = totc * CHUNK
    src_p = jnp.concatenate(
        [src, jnp.zeros((e_pad - e,), jnp.int32)]).reshape(totc, CHUNK)
    dst_p = jnp.concatenate(
        [dst, jnp.full((e_pad - e,), n, jnp.int32)]).reshape(totc, CHUNK)

    degp = _deg_partials(dst_p)

    degp_spec = pl.BlockSpec((N_SC, bm, 128), lambda i: (0, i, 0))
    y1 = pl.pallas_call(
        _k1_body,
        grid=grid,
        in_specs=[
            pl.BlockSpec((bm, x.shape[1]), lambda i: (i, 0)),
            pl.BlockSpec(W1.shape, lambda i: (0, 0)),
            degp_spec,
        ],
        out_specs=pl.BlockSpec((bm, hidden), lambda i: (i, 0)),
        out_shape=jax.ShapeDtypeStruct((n, hidden), jnp.float32),
    )(x, W1, degp)

    p = _aggregate(y1, src_p, dst_p)

    y2 = pl.pallas_call(
        _k2_body,
        grid=grid,
        in_specs=[
            pl.BlockSpec((N_SC, bm, hidden), lambda i: (0, i, 0)),
            pl.BlockSpec((bm, hidden), lambda i: (i, 0)),
            degp_spec,
            pl.BlockSpec((1, hidden), lambda i: (0, 0)),
            pl.BlockSpec(W2.shape, lambda i: (0, 0)),
        ],
        out_specs=pl.BlockSpec((bm, 2 * classes), lambda i: (i, 0)),
        out_shape=jax.ShapeDtypeStruct((n, 2 * classes), jnp.float32),
    )(p, y1, degp, b1.reshape(1, -1), W2)

    q = _aggregate(y2, src_p, dst_p)

    out = pl.pallas_call(
        _k3_body,
        grid=grid,
        in_specs=[
            pl.BlockSpec((N_SC, bm, 2 * classes), lambda i: (0, i, 0)),
            pl.BlockSpec((bm, 2 * classes), lambda i: (i, 0)),
            degp_spec,
            pl.BlockSpec((1, classes), lambda i: (0, 0)),
        ],
        out_specs=pl.BlockSpec((bm, classes), lambda i: (i, 0)),
        out_shape=jax.ShapeDtypeStruct((n, classes), jnp.float32),
    )(q, y2, degp, b2.reshape(1, -1))
    return out


# speedup vs baseline: 3.0541x; 2.4602x over previous
import functools

import jax
import jax.numpy as jnp
from jax import lax
from jax.experimental import pallas as pl
from jax.experimental.pallas import tpu as pltpu
from jax.experimental.pallas import tpu_sc as plsc

N_SC = 2
N_TILES = 16
NW = N_SC * N_TILES
CHUNK = 128
GRP = 16
NPAD = 10112
M0 = 160
M1 = 160
MTOT = M0 + M1


def _k1_body(x_ref, w_ref, degp_ref, o_ref):
    deg = degp_ref[0, :, 0] + degp_ref[1, :, 0] + 1.0
    dis = lax.rsqrt(deg)
    xw = jnp.dot(x_ref[...], w_ref[...], preferred_element_type=jnp.float32)
    o_ref[...] = dis[:, None] * xw


def _k2_body(p_ref, y1_ref, degp_ref, b1_ref, w2_ref, o_ref):
    deg = degp_ref[0, :, 0] + degp_ref[1, :, 0] + 1.0
    dis = lax.rsqrt(deg)
    s = p_ref[0] + p_ref[1] + y1_ref[...]
    h = jnp.maximum(dis[:, None] * s + b1_ref[...], 0.0)
    hw = jnp.dot(h, w2_ref[...], preferred_element_type=jnp.float32)
    o_ref[...] = jnp.concatenate(
        [dis[:, None] * hw, jnp.zeros_like(hw)], axis=1)


def _k3_body(q_ref, y2_ref, degp_ref, b2_ref, o_ref):
    deg = degp_ref[0, :, 0] + degp_ref[1, :, 0] + 1.0
    dis = lax.rsqrt(deg)
    nc = o_ref.shape[1]
    o = (dis[:, None] * (q_ref[0, :, :nc] + q_ref[1, :, :nc] + y2_ref[:, :nc])
         + b2_ref[...])
    m = jnp.max(o, axis=1, keepdims=True)
    lse = jnp.log(jnp.sum(jnp.exp(o - m), axis=1, keepdims=True)) + m
    o_ref[...] = o - lse


def _deg_body(dst_hbm, ones_hbm, zeros_hbm, out_hbm, dst_g, ones_v, deg_sh):
    c = lax.axis_index("c")
    s = lax.axis_index("s")
    w = c * N_TILES + s
    mpw = dst_hbm.shape[0] // NW
    off = w * mpw
    pltpu.sync_copy(ones_hbm, ones_v)
    pltpu.sync_copy(zeros_hbm, deg_sh.at[pl.ds(s * (NPAD // N_TILES), NPAD // N_TILES)])
    plsc.subcore_barrier()

    def group(g, carry):
        pltpu.sync_copy(dst_hbm.at[pl.ds(off + g * GRP, GRP)], dst_g)
        for j in range(GRP):
            pltpu.sync_copy(ones_v, deg_sh.at[dst_g.at[j]], add=True)
        return carry

    lax.fori_loop(0, mpw // GRP, group, 0, unroll=False)
    plsc.subcore_barrier()
    npt = NPAD // N_TILES
    pltpu.sync_copy(deg_sh.at[pl.ds(s * npt, npt)], out_hbm.at[c, pl.ds(s * npt, npt)])


def _agg_body(y_hbm, src_hbm, dst_hbm, zeros_hbm, out_hbm,
              src_g, dst_g, rows0, rows1, agg_sh,
              gsem0, gsem1, ssem0, ssem1):
    c = lax.axis_index("c")
    s = lax.axis_index("s")
    rows = (rows0, rows1)
    gsem = (gsem0, gsem1)
    ssem = (ssem0, ssem1)
    pltpu.sync_copy(zeros_hbm, agg_sh.at[pl.ds(s * (NPAD // N_TILES), NPAD // N_TILES)])
    plsc.subcore_barrier()

    def run(off, ngroups):
        def group(g, carry):
            base = off + g * GRP
            pltpu.sync_copy(src_hbm.at[pl.ds(base, GRP)], src_g)
            pltpu.sync_copy(dst_hbm.at[pl.ds(base, GRP)], dst_g)
            gd = [None, None]
            sd = [None, None]
            gd[0] = pltpu.async_copy(y_hbm.at[src_g.at[0]], rows[0], gsem[0])
            for j in range(GRP):
                b = j & 1
                nb = 1 - b
                if j + 1 < GRP:
                    if sd[nb] is not None:
                        sd[nb].wait()
                    gd[nb] = pltpu.async_copy(
                        y_hbm.at[src_g.at[j + 1]], rows[nb], gsem[nb])
                gd[b].wait()
                sd[b] = pltpu.async_copy(
                    rows[b], agg_sh.at[dst_g.at[j]], ssem[b], add=True)
            sd[(GRP - 2) & 1].wait()
            sd[(GRP - 1) & 1].wait()
            return carry
        lax.fori_loop(0, ngroups, group, 0, unroll=False)

    @pl.when(c == 0)
    def _():
        run(s * M0, M0 // GRP)

    @pl.when(c == 1)
    def _():
        run(N_TILES * M0 + s * M1, M1 // GRP)

    plsc.subcore_barrier()
    npt = NPAD // N_TILES
    pltpu.sync_copy(agg_sh.at[pl.ds(s * npt, npt)], out_hbm.at[c, pl.ds(s * npt, npt)])


@functools.cache
def _sc_mesh():
    return plsc.VectorSubcoreMesh(
        core_axis_name="c", subcore_axis_name="s",
        num_cores=N_SC, num_subcores=N_TILES)


def _deg_partials(dst_p):
    ones_rows = jnp.zeros((CHUNK, 128), jnp.float32).at[:, 0].set(1.0)
    zeros = jnp.zeros((NPAD // N_TILES, 128), jnp.float32)
    return pl.kernel(
        _deg_body,
        out_type=jax.ShapeDtypeStruct((N_SC, NPAD, 128), jnp.float32),
        mesh=_sc_mesh(),
        scratch_types=[
            pltpu.VMEM((GRP, CHUNK), jnp.int32),
            pltpu.VMEM((CHUNK, 128), jnp.float32),
            pltpu.VMEM_SHARED((NPAD, 128), jnp.float32),
        ],
    )(dst_p, ones_rows, zeros)


def _aggregate(y, src_p, dst_p):
    d = y.shape[1]
    zeros = jnp.zeros((NPAD // N_TILES, d), jnp.float32)
    return pl.kernel(
        _agg_body,
        out_type=jax.ShapeDtypeStruct((N_SC, NPAD, d), jnp.float32),
        mesh=_sc_mesh(),
        scratch_types=[
            pltpu.VMEM((GRP, CHUNK), jnp.int32),
            pltpu.VMEM((GRP, CHUNK), jnp.int32),
            pltpu.VMEM((CHUNK, d), jnp.float32),
            pltpu.VMEM((CHUNK, d), jnp.float32),
            pltpu.VMEM_SHARED((NPAD, d), jnp.float32),
            pltpu.SemaphoreType.DMA,
            pltpu.SemaphoreType.DMA,
            pltpu.SemaphoreType.DMA,
            pltpu.SemaphoreType.DMA,
        ],
    )(y, src_p, dst_p, zeros)


def kernel(x, edge_index, W1, b1, W2, b2):
    n, _ = x.shape
    hidden = W1.shape[1]
    classes = W2.shape[1]
    e = edge_index.shape[1]
    bm = 400
    grid = (n // bm,)

    src = edge_index[0].astype(jnp.int32)
    dst = edge_index[1].astype(jnp.int32)
    totc = N_TILES * MTOT
    e_pad = totc * CHUNK
    pad_src = jnp.arange(e_pad - e, dtype=jnp.int32) % n
    src_p = jnp.concatenate([src, pad_src]).reshape(totc, CHUNK)
    dst_p = jnp.concatenate(
        [dst, jnp.full((e_pad - e,), n, jnp.int32)]).reshape(totc, CHUNK)

    degp = _deg_partials(dst_p)

    degp_spec = pl.BlockSpec((N_SC, bm, 128), lambda i: (0, i, 0))
    y1 = pl.pallas_call(
        _k1_body,
        grid=grid,
        in_specs=[
            pl.BlockSpec((bm, x.shape[1]), lambda i: (i, 0)),
            pl.BlockSpec(W1.shape, lambda i: (0, 0)),
            degp_spec,
        ],
        out_specs=pl.BlockSpec((bm, hidden), lambda i: (i, 0)),
        out_shape=jax.ShapeDtypeStruct((n, hidden), jnp.float32),
    )(x, W1, degp)

    p = _aggregate(y1, src_p, dst_p)

    y2 = pl.pallas_call(
        _k2_body,
        grid=grid,
        in_specs=[
            pl.BlockSpec((N_SC, bm, hidden), lambda i: (0, i, 0)),
            pl.BlockSpec((bm, hidden), lambda i: (i, 0)),
            degp_spec,
            pl.BlockSpec((1, hidden), lambda i: (0, 0)),
            pl.BlockSpec(W2.shape, lambda i: (0, 0)),
        ],
        out_specs=pl.BlockSpec((bm, 2 * classes), lambda i: (i, 0)),
        out_shape=jax.ShapeDtypeStruct((n, 2 * classes), jnp.float32),
    )(p, y1, degp, b1.reshape(1, -1), W2)

    q = _aggregate(y2, src_p, dst_p)

    out = pl.pallas_call(
        _k3_body,
        grid=grid,
        in_specs=[
            pl.BlockSpec((N_SC, bm, 2 * classes), lambda i: (0, i, 0)),
            pl.BlockSpec((bm, 2 * classes), lambda i: (i, 0)),
            degp_spec,
            pl.BlockSpec((1, classes), lambda i: (0, 0)),
        ],
        out_specs=pl.BlockSpec((bm, classes), lambda i: (i, 0)),
        out_shape=jax.ShapeDtypeStruct((n, classes), jnp.float32),
    )(q, y2, degp, b2.reshape(1, -1))
    return out


# speedup vs baseline: 3.1908x; 1.0448x over previous
import functools

import jax
import jax.numpy as jnp
from jax import lax
from jax.experimental import pallas as pl
from jax.experimental.pallas import tpu as pltpu
from jax.experimental.pallas import tpu_sc as plsc

N_SC = 2
N_TILES = 16
NW = N_SC * N_TILES
CHUNK = 128
GRP = 16
NPAD = 10112
NPADB = 10240
M0 = 160
M1 = 160
MTOT = M0 + M1


def _k1a_body(x_ref, w_ref, o_ref):
    o_ref[...] = jnp.dot(x_ref[...], w_ref[...],
                         preferred_element_type=jnp.float32)


def _k1b_body(xw_ref, degp_ref, o_ref):
    deg = degp_ref[0, :, 0] + degp_ref[1, :, 0] + 1.0
    dis = lax.rsqrt(deg)
    o_ref[...] = dis[:, None] * xw_ref[...]


def _k2_body(p_ref, y1_ref, degp_ref, b1_ref, w2_ref, o_ref):
    deg = degp_ref[0, :, 0] + degp_ref[1, :, 0] + 1.0
    dis = lax.rsqrt(deg)
    s = p_ref[0] + p_ref[1] + y1_ref[...]
    h = jnp.maximum(dis[:, None] * s + b1_ref[...], 0.0)
    hw = jnp.dot(h, w2_ref[...], preferred_element_type=jnp.float32)
    o_ref[...] = jnp.concatenate(
        [dis[:, None] * hw, jnp.zeros_like(hw)], axis=1)


def _k3_body(q_ref, y2_ref, degp_ref, b2_ref, o_ref):
    deg = degp_ref[0, :, 0] + degp_ref[1, :, 0] + 1.0
    dis = lax.rsqrt(deg)
    nc = o_ref.shape[1]
    o = (dis[:, None] * (q_ref[0, :, :nc] + q_ref[1, :, :nc] + y2_ref[:, :nc])
         + b2_ref[...])
    m = jnp.max(o, axis=1, keepdims=True)
    lse = jnp.log(jnp.sum(jnp.exp(o - m), axis=1, keepdims=True)) + m
    o_ref[...] = o - lse


def _deg_body(dst_hbm, ones_hbm, zeros_hbm, out_hbm, dst_g, ones_v, deg_sh):
    c = lax.axis_index("c")
    s = lax.axis_index("s")
    w = c * N_TILES + s
    mpw = dst_hbm.shape[0] // NW
    off = w * mpw
    npt = NPADB // N_TILES
    pltpu.sync_copy(ones_hbm, ones_v)
    pltpu.sync_copy(zeros_hbm, deg_sh.at[pl.ds(s * npt, npt)])
    plsc.subcore_barrier()

    def group(g, carry):
        pltpu.sync_copy(dst_hbm.at[pl.ds(off + g * GRP, GRP)], dst_g)
        for j in range(GRP):
            pltpu.sync_copy(ones_v, deg_sh.at[dst_g.at[j]], add=True)
        return carry

    lax.fori_loop(0, mpw // GRP, group, 0, unroll=False)
    plsc.subcore_barrier()
    pltpu.sync_copy(deg_sh.at[pl.ds(s * npt, npt)], out_hbm.at[c, pl.ds(s * npt, npt)])


def _agg_body(y_hbm, src_hbm, dst_hbm, zeros_hbm, out_hbm,
              src_g, dst_g, rows0, rows1, agg_sh,
              gsem0, gsem1, ssem0, ssem1):
    c = lax.axis_index("c")
    s = lax.axis_index("s")
    rows = (rows0, rows1)
    gsem = (gsem0, gsem1)
    ssem = (ssem0, ssem1)
    pltpu.sync_copy(zeros_hbm, agg_sh.at[pl.ds(s * (NPAD // N_TILES), NPAD // N_TILES)])
    plsc.subcore_barrier()

    def run(off, ngroups):
        def group(g, carry):
            base = off + g * GRP
            pltpu.sync_copy(src_hbm.at[pl.ds(base, GRP)], src_g)
            pltpu.sync_copy(dst_hbm.at[pl.ds(base, GRP)], dst_g)
            gd = [None, None]
            sd = [None, None]
            gd[0] = pltpu.async_copy(y_hbm.at[src_g.at[0]], rows[0], gsem[0])
            for j in range(GRP):
                b = j & 1
                nb = 1 - b
                if j + 1 < GRP:
                    if sd[nb] is not None:
                        sd[nb].wait()
                    gd[nb] = pltpu.async_copy(
                        y_hbm.at[src_g.at[j + 1]], rows[nb], gsem[nb])
                gd[b].wait()
                sd[b] = pltpu.async_copy(
                    rows[b], agg_sh.at[dst_g.at[j]], ssem[b], add=True)
            sd[(GRP - 2) & 1].wait()
            sd[(GRP - 1) & 1].wait()
            return carry
        lax.fori_loop(0, ngroups, group, 0, unroll=False)

    @pl.when(c == 0)
    def _():
        run(s * M0, M0 // GRP)

    @pl.when(c == 1)
    def _():
        run(N_TILES * M0 + s * M1, M1 // GRP)

    plsc.subcore_barrier()
    npt = NPAD // N_TILES
    pltpu.sync_copy(agg_sh.at[pl.ds(s * npt, npt)], out_hbm.at[c, pl.ds(s * npt, npt)])


@functools.cache
def _sc_mesh():
    return plsc.VectorSubcoreMesh(
        core_axis_name="c", subcore_axis_name="s",
        num_cores=N_SC, num_subcores=N_TILES)


def _deg_partials(dst_p):
    ones_rows = jnp.zeros((CHUNK, 128), jnp.float32).at[:, 0].set(1.0)
    zeros = jnp.zeros((NPADB // N_TILES, 128), jnp.float32)
    return pl.kernel(
        _deg_body,
        out_type=jax.ShapeDtypeStruct((N_SC, NPADB, 128), jnp.float32),
        mesh=_sc_mesh(),
        scratch_types=[
            pltpu.VMEM((GRP, CHUNK), jnp.int32),
            pltpu.VMEM((CHUNK, 128), jnp.float32),
            pltpu.VMEM_SHARED((NPADB, 128), jnp.float32),
        ],
    )(dst_p, ones_rows, zeros)


def _aggregate(y, src_p, dst_p):
    d = y.shape[1]
    zeros = jnp.zeros((NPAD // N_TILES, d), jnp.float32)
    return pl.kernel(
        _agg_body,
        out_type=jax.ShapeDtypeStruct((N_SC, NPAD, d), jnp.float32),
        mesh=_sc_mesh(),
        scratch_types=[
            pltpu.VMEM((GRP, CHUNK), jnp.int32),
            pltpu.VMEM((GRP, CHUNK), jnp.int32),
            pltpu.VMEM((CHUNK, d), jnp.float32),
            pltpu.VMEM((CHUNK, d), jnp.float32),
            pltpu.VMEM_SHARED((NPAD, d), jnp.float32),
            pltpu.SemaphoreType.DMA,
            pltpu.SemaphoreType.DMA,
            pltpu.SemaphoreType.DMA,
            pltpu.SemaphoreType.DMA,
        ],
    )(y, src_p, dst_p, zeros)


def kernel(x, edge_index, W1, b1, W2, b2):
    n, _ = x.shape
    hidden = W1.shape[1]
    classes = W2.shape[1]
    e = edge_index.shape[1]
    bm = 400
    grid = (n // bm,)

    src = edge_index[0].astype(jnp.int32)
    dst = edge_index[1].astype(jnp.int32)
    totc = N_TILES * MTOT
    e_pad = totc * CHUNK
    pad_src = jnp.arange(e_pad - e, dtype=jnp.int32) % n
    src_p = jnp.concatenate([src, pad_src]).reshape(totc, CHUNK)
    dst_p = jnp.concatenate(
        [dst, jnp.full((e_pad - e,), n, jnp.int32)]).reshape(totc, CHUNK)

    degp = _deg_partials(dst_p)

    degp_spec = pl.BlockSpec((N_SC, bm, 128), lambda i: (0, i, 0))
    xw = pl.pallas_call(
        _k1a_body,
        grid=grid,
        in_specs=[
            pl.BlockSpec((bm, x.shape[1]), lambda i: (i, 0)),
            pl.BlockSpec(W1.shape, lambda i: (0, 0)),
        ],
        out_specs=pl.BlockSpec((bm, hidden), lambda i: (i, 0)),
        out_shape=jax.ShapeDtypeStruct((n, hidden), jnp.float32),
    )(x, W1)
    y1 = pl.pallas_call(
        _k1b_body,
        grid=grid,
        in_specs=[
            pl.BlockSpec((bm, hidden), lambda i: (i, 0)),
            degp_spec,
        ],
        out_specs=pl.BlockSpec((bm, hidden), lambda i: (i, 0)),
        out_shape=jax.ShapeDtypeStruct((n, hidden), jnp.float32),
    )(xw, degp)

    p = _aggregate(y1, src_p, dst_p)

    y2 = pl.pallas_call(
        _k2_body,
        grid=grid,
        in_specs=[
            pl.BlockSpec((N_SC, bm, hidden), lambda i: (0, i, 0)),
            pl.BlockSpec((bm, hidden), lambda i: (i, 0)),
            degp_spec,
            pl.BlockSpec((1, hidden), lambda i: (0, 0)),
            pl.BlockSpec(W2.shape, lambda i: (0, 0)),
        ],
        out_specs=pl.BlockSpec((bm, 2 * classes), lambda i: (i, 0)),
        out_shape=jax.ShapeDtypeStruct((n, 2 * classes), jnp.float32),
    )(p, y1, degp, b1.reshape(1, -1), W2)

    q = _aggregate(y2, src_p, dst_p)

    out = pl.pallas_call(
        _k3_body,
        grid=grid,
        in_specs=[
            pl.BlockSpec((N_SC, bm, 2 * classes), lambda i: (0, i, 0)),
            pl.BlockSpec((bm, 2 * classes), lambda i: (i, 0)),
            degp_spec,
            pl.BlockSpec((1, classes), lambda i: (0, 0)),
        ],
        out_specs=pl.BlockSpec((bm, classes), lambda i: (i, 0)),
        out_shape=jax.ShapeDtypeStruct((n, classes), jnp.float32),
    )(q, y2, degp, b2.reshape(1, -1))
    return out


# speedup vs baseline: 3.3267x; 1.0426x over previous
import functools

import jax
import jax.numpy as jnp
from jax import lax
from jax.experimental import pallas as pl
from jax.experimental.pallas import tpu as pltpu
from jax.experimental.pallas import tpu_sc as plsc

N_SC = 2
N_TILES = 16
NW = N_SC * N_TILES
CHUNK = 128
GRP = 32
NPAD = 10112
NPADB = 10240
M0 = 160
M1 = 160
MTOT = M0 + M1


def _k1a_body(x_ref, w_ref, o_ref):
    o_ref[...] = jnp.dot(x_ref[...], w_ref[...],
                         preferred_element_type=jnp.float32)


def _k1b_body(xw_ref, degp_ref, o_ref):
    deg = degp_ref[0, :, 0] + degp_ref[1, :, 0] + 1.0
    dis = lax.rsqrt(deg)
    o_ref[...] = dis[:, None] * xw_ref[...]


def _k2_body(p_ref, y1_ref, degp_ref, b1_ref, w2_ref, o_ref):
    deg = degp_ref[0, :, 0] + degp_ref[1, :, 0] + 1.0
    dis = lax.rsqrt(deg)
    s = p_ref[0] + p_ref[1] + y1_ref[...]
    h = jnp.maximum(dis[:, None] * s + b1_ref[...], 0.0)
    hw = jnp.dot(h, w2_ref[...], preferred_element_type=jnp.float32)
    o_ref[...] = jnp.concatenate(
        [dis[:, None] * hw, jnp.zeros_like(hw)], axis=1)


def _k3_body(q_ref, y2_ref, degp_ref, b2_ref, o_ref):
    deg = degp_ref[0, :, 0] + degp_ref[1, :, 0] + 1.0
    dis = lax.rsqrt(deg)
    nc = o_ref.shape[1]
    o = (dis[:, None] * (q_ref[0, :, :nc] + q_ref[1, :, :nc] + y2_ref[:, :nc])
         + b2_ref[...])
    m = jnp.max(o, axis=1, keepdims=True)
    lse = jnp.log(jnp.sum(jnp.exp(o - m), axis=1, keepdims=True)) + m
    o_ref[...] = o - lse


def _deg_body(dst_hbm, ones_hbm, zeros_hbm, out_hbm, dst_g, ones_v, deg_sh):
    c = lax.axis_index("c")
    s = lax.axis_index("s")
    w = c * N_TILES + s
    mpw = dst_hbm.shape[0] // NW
    off = w * mpw
    npt = NPADB // N_TILES
    pltpu.sync_copy(ones_hbm, ones_v)
    pltpu.sync_copy(zeros_hbm, deg_sh.at[pl.ds(s * npt, npt)])
    plsc.subcore_barrier()

    def group(g, carry):
        pltpu.sync_copy(dst_hbm.at[pl.ds(off + g * GRP, GRP)], dst_g)
        for j in range(GRP):
            pltpu.sync_copy(ones_v, deg_sh.at[dst_g.at[j]], add=True)
        return carry

    lax.fori_loop(0, mpw // GRP, group, 0, unroll=False)
    plsc.subcore_barrier()
    pltpu.sync_copy(deg_sh.at[pl.ds(s * npt, npt)], out_hbm.at[c, pl.ds(s * npt, npt)])


def _agg_body(y_hbm, src_hbm, dst_hbm, zeros_hbm, out_hbm,
              src_g, dst_g, rows0, rows1, agg_sh,
              gsem0, gsem1, ssem0, ssem1):
    c = lax.axis_index("c")
    s = lax.axis_index("s")
    rows = (rows0, rows1)
    gsem = (gsem0, gsem1)
    ssem = (ssem0, ssem1)
    pltpu.sync_copy(zeros_hbm, agg_sh.at[pl.ds(s * (NPAD // N_TILES), NPAD // N_TILES)])
    plsc.subcore_barrier()

    def run(off, ngroups):
        def group(g, carry):
            base = off + g * GRP
            pltpu.sync_copy(src_hbm.at[pl.ds(base, GRP)], src_g)
            pltpu.sync_copy(dst_hbm.at[pl.ds(base, GRP)], dst_g)
            gd = [None, None]
            sd = [None, None]
            gd[0] = pltpu.async_copy(y_hbm.at[src_g.at[0]], rows[0], gsem[0])
            for j in range(GRP):
                b = j & 1
                nb = 1 - b
                if j + 1 < GRP:
                    if sd[nb] is not None:
                        sd[nb].wait()
                    gd[nb] = pltpu.async_copy(
                        y_hbm.at[src_g.at[j + 1]], rows[nb], gsem[nb])
                gd[b].wait()
                sd[b] = pltpu.async_copy(
                    rows[b], agg_sh.at[dst_g.at[j]], ssem[b], add=True)
            sd[(GRP - 2) & 1].wait()
            sd[(GRP - 1) & 1].wait()
            return carry
        lax.fori_loop(0, ngroups, group, 0, unroll=False)

    @pl.when(c == 0)
    def _():
        run(s * M0, M0 // GRP)

    @pl.when(c == 1)
    def _():
        run(N_TILES * M0 + s * M1, M1 // GRP)

    plsc.subcore_barrier()
    npt = NPAD // N_TILES
    pltpu.sync_copy(agg_sh.at[pl.ds(s * npt, npt)], out_hbm.at[c, pl.ds(s * npt, npt)])


@functools.cache
def _sc_mesh():
    return plsc.VectorSubcoreMesh(
        core_axis_name="c", subcore_axis_name="s",
        num_cores=N_SC, num_subcores=N_TILES)


def _deg_partials(dst_p):
    ones_rows = jnp.zeros((CHUNK, 128), jnp.float32).at[:, 0].set(1.0)
    zeros = jnp.zeros((NPADB // N_TILES, 128), jnp.float32)
    return pl.kernel(
        _deg_body,
        out_type=jax.ShapeDtypeStruct((N_SC, NPADB, 128), jnp.float32),
        mesh=_sc_mesh(),
        scratch_types=[
            pltpu.VMEM((GRP, CHUNK), jnp.int32),
            pltpu.VMEM((CHUNK, 128), jnp.float32),
            pltpu.VMEM_SHARED((NPADB, 128), jnp.float32),
        ],
    )(dst_p, ones_rows, zeros)


def _aggregate(y, src_p, dst_p):
    d = y.shape[1]
    zeros = jnp.zeros((NPAD // N_TILES, d), jnp.float32)
    return pl.kernel(
        _agg_body,
        out_type=jax.ShapeDtypeStruct((N_SC, NPAD, d), jnp.float32),
        mesh=_sc_mesh(),
        scratch_types=[
            pltpu.VMEM((GRP, CHUNK), jnp.int32),
            pltpu.VMEM((GRP, CHUNK), jnp.int32),
            pltpu.VMEM((CHUNK, d), jnp.float32),
            pltpu.VMEM((CHUNK, d), jnp.float32),
            pltpu.VMEM_SHARED((NPAD, d), jnp.float32),
            pltpu.SemaphoreType.DMA,
            pltpu.SemaphoreType.DMA,
            pltpu.SemaphoreType.DMA,
            pltpu.SemaphoreType.DMA,
        ],
    )(y, src_p, dst_p, zeros)


def kernel(x, edge_index, W1, b1, W2, b2):
    n, _ = x.shape
    hidden = W1.shape[1]
    classes = W2.shape[1]
    e = edge_index.shape[1]
    bm = 400
    grid = (n // bm,)

    src = edge_index[0].astype(jnp.int32)
    dst = edge_index[1].astype(jnp.int32)
    totc = N_TILES * MTOT
    e_pad = totc * CHUNK
    pad_src = jnp.arange(e_pad - e, dtype=jnp.int32) % n
    src_p = jnp.concatenate([src, pad_src]).reshape(totc, CHUNK)
    dst_p = jnp.concatenate(
        [dst, jnp.full((e_pad - e,), n, jnp.int32)]).reshape(totc, CHUNK)

    degp = _deg_partials(dst_p)

    degp_spec = pl.BlockSpec((N_SC, bm, 128), lambda i: (0, i, 0))
    xw = pl.pallas_call(
        _k1a_body,
        grid=grid,
        in_specs=[
            pl.BlockSpec((bm, x.shape[1]), lambda i: (i, 0)),
            pl.BlockSpec(W1.shape, lambda i: (0, 0)),
        ],
        out_specs=pl.BlockSpec((bm, hidden), lambda i: (i, 0)),
        out_shape=jax.ShapeDtypeStruct((n, hidden), jnp.float32),
    )(x, W1)
    y1 = pl.pallas_call(
        _k1b_body,
        grid=grid,
        in_specs=[
            pl.BlockSpec((bm, hidden), lambda i: (i, 0)),
            degp_spec,
        ],
        out_specs=pl.BlockSpec((bm, hidden), lambda i: (i, 0)),
        out_shape=jax.ShapeDtypeStruct((n, hidden), jnp.float32),
    )(xw, degp)

    p = _aggregate(y1, src_p, dst_p)

    y2 = pl.pallas_call(
        _k2_body,
        grid=grid,
        in_specs=[
            pl.BlockSpec((N_SC, bm, hidden), lambda i: (0, i, 0)),
            pl.BlockSpec((bm, hidden), lambda i: (i, 0)),
            degp_spec,
            pl.BlockSpec((1, hidden), lambda i: (0, 0)),
            pl.BlockSpec(W2.shape, lambda i: (0, 0)),
        ],
        out_specs=pl.BlockSpec((bm, 2 * classes), lambda i: (i, 0)),
        out_shape=jax.ShapeDtypeStruct((n, 2 * classes), jnp.float32),
    )(p, y1, degp, b1.reshape(1, -1), W2)

    q = _aggregate(y2, src_p, dst_p)

    out = pl.pallas_call(
        _k3_body,
        grid=grid,
        in_specs=[
            pl.BlockSpec((N_SC, bm, 2 * classes), lambda i: (0, i, 0)),
            pl.BlockSpec((bm, 2 * classes), lambda i: (i, 0)),
            degp_spec,
            pl.BlockSpec((1, classes), lambda i: (0, 0)),
        ],
        out_specs=pl.BlockSpec((bm, classes), lambda i: (i, 0)),
        out_shape=jax.ShapeDtypeStruct((n, classes), jnp.float32),
    )(q, y2, degp, b2.reshape(1, -1))
    return out
